# baseline (speedup 1.0000x reference)
import functools

import jax
import jax.numpy as jnp
from jax import lax
from jax.experimental import pallas as pl
from jax.experimental.pallas import tpu as pltpu
from jax.experimental.pallas import tpu_sc as plsc

_K_CL = 5
_K_PAD = 8
_MAX_ITER = 100


def _gram_body(x_ref, w_ref, b_ref, k_ref):
    cb, n, d = x_ref.shape
    x2d = x_ref[...].reshape(cb * n, d)
    emb = (
        jnp.dot(x2d, w_ref[...], preferred_element_type=jnp.float32)
        + b_ref[...]
    )
    for c in range(cb):
        ec = emb[c * n : (c + 1) * n]
        k_ref[c] = lax.dot_general(
            ec, ec, (((1,), (1,)), ((), ())), preferred_element_type=jnp.float32
        )


def _gram(x, W, b, class_block=8):
    n_way, n, d = x.shape
    return pl.pallas_call(
        _gram_body,
        grid=(n_way // class_block,),
        in_specs=[
            pl.BlockSpec((class_block, n, d), lambda i: (i, 0, 0)),
            pl.BlockSpec((d, d), lambda i: (0, 0)),
            pl.BlockSpec((1, d), lambda i: (0, 0)),
        ],
        out_specs=pl.BlockSpec((class_block, n, n), lambda i: (i, 0, 0)),
        out_shape=jax.ShapeDtypeStruct((n_way, n, n), jnp.float32),
    )(x, W, b.reshape(1, d))


def _kmeans_body(k_ref, w0_ref, u_ref, w_ref, prev_ref):
    n_way, kp, n = w0_ref.shape
    w_ref[...] = w0_ref[...]
    prev_ref[...] = jnp.full((n_way, n), -1, jnp.int32)

    jidx = lax.broadcasted_iota(jnp.int32, (kp, n), 0)
    jbad = jidx >= _K_CL

    def one_class(c, carry):
        kc = k_ref[c]

        def cond(it_ch):
            it, changed = it_ch
            return jnp.logical_and(it < _MAX_ITER, changed)

        def body(it_ch):
            it, _ = it_ch
            wc = w_ref[c]
            g = lax.dot_general(
                wc, kc, (((1,), (0,)), ((), ())),
                preferred_element_type=jnp.float32,
            )
            c2 = jnp.sum(g * wc, axis=1, keepdims=True)
            score = c2 - 2.0 * g
            score = jnp.where(jbad, jnp.inf, score)
            smin = jnp.min(score, axis=0, keepdims=True)
            assign = jnp.min(
                jnp.where(score == smin, jidx, _K_PAD), axis=0, keepdims=True
            )
            onehot = (jidx == assign).astype(jnp.float32)
            counts = jnp.sum(onehot, axis=1, keepdims=True)
            w_ref[c] = jnp.where(counts > 0.0, onehot / counts, wc)
            ch = jnp.any(assign[0] != prev_ref[c])
            prev_ref[c] = assign[0]
            return it + 1, ch

        lax.while_loop(cond, body, (jnp.int32(0), jnp.bool_(True)))
        return carry

    lax.fori_loop(0, n_way, one_class, jnp.int32(0))
    u_ref[...] = jnp.sum(w_ref[...], axis=1) * (1.0 / _K_CL)


def _kmeans(K, w0):
    n_way, kp, n = w0.shape
    return pl.pallas_call(
        _kmeans_body,
        in_specs=[
            pl.BlockSpec((n_way, n, n), lambda: (0, 0, 0)),
            pl.BlockSpec((n_way, kp, n), lambda: (0, 0, 0)),
        ],
        out_specs=pl.BlockSpec((n_way, n), lambda: (0, 0)),
        out_shape=jax.ShapeDtypeStruct((n_way, n), jnp.float32),
        scratch_shapes=[
            pltpu.VMEM((n_way, kp, n), jnp.float32),
            pltpu.VMEM((n_way, n), jnp.int32),
        ],
    )(K, w0)


def _sc_kmeans(K, idxp):
    n_way, n, _ = K.shape
    L = 16
    nch = n // L
    half = n // 2
    NC, NS = 2, 16
    NW = NC * NS
    per_w = n_way // NW
    mesh = plsc.VectorSubcoreMesh(
        core_axis_name="c", subcore_axis_name="s",
        num_cores=NC, num_subcores=NS,
    )
    f32, i32 = jnp.float32, jnp.int32

    @functools.partial(
        pl.kernel,
        out_type=jax.ShapeDtypeStruct((n_way, n), f32),
        mesh=mesh,
        compiler_params=pltpu.CompilerParams(needs_layout_passes=False),
        scratch_types=[
            pltpu.VMEM((n, n), f32),
            pltpu.VMEM((16, n), f32),
            pltpu.VMEM((16, n), f32),
            pltpu.VMEM((8, n), f32),
            pltpu.VMEM((n,), i32),
            pltpu.VMEM((8, L), f32),
            pltpu.VMEM((8, L), f32),
            pltpu.VMEM((L,), i32),
            pltpu.VMEM((n,), f32),
            pltpu.VMEM((L,), f32),
        ],
    )
    def body(k_hbm, idx_hbm, u_hbm, kbuf, sacc, snew, wgt,
             prev, cinv, c2b, idxv, ubuf, t16):
        wid = lax.axis_index("s") * NC + lax.axis_index("c")
        lane = lax.iota(i32, L)

        def run_class(r, _carry):
            cls = wid * per_w + r
            pltpu.sync_copy(k_hbm.at[cls], kbuf)
            pltpu.sync_copy(idx_hbm.at[cls], idxv)
            for j in range(_K_CL):
                js = jnp.full((L,), j, i32)
                isp = plsc.load_gather(idxv, [js])
                cinv[j] = jnp.ones((L,), f32)
                c2b[j] = plsc.load_gather(kbuf, [isp, isp])
                for cc in range(nch):
                    ids = lane + (cc * L)
                    sacc[j, pl.ds(cc * L, L)] = plsc.load_gather(
                        kbuf, [isp, ids]
                    )
                    wgt[j, pl.ds(cc * L, L)] = jnp.where(ids == isp, 1.0, 0.0)
            for cc in range(nch):
                prev[pl.ds(cc * L, L)] = jnp.full((L,), -1, i32)

            def cond(carry):
                it, changed = carry
                return jnp.logical_and(it < _MAX_ITER, changed)

            def step(carry):
                it, _ = carry
                m2 = [cinv[j] * -2.0 for j in range(_K_CL)]
                c2 = [c2b[j][...] for j in range(_K_CL)]
                cnt = [jnp.zeros((L,), i32) for _ in range(_K_CL)]
                changed = jnp.zeros((L,), jnp.bool_)
                for cc in range(nch):
                    sl = pl.ds(cc * L, L)
                    best = jnp.full((L,), jnp.inf, f32)
                    bidx = jnp.full((L,), _K_CL, i32)
                    for j in range(_K_CL):
                        sc = c2[j] + m2[j] * sacc[j, sl]
                        lt = sc < best
                        best = jnp.where(lt, sc, best)
                        bidx = jnp.where(lt, j, bidx)
                    changed = jnp.logical_or(changed, bidx != prev[sl])
                    prev[sl] = bidx
                    for j in range(_K_CL):
                        cnt[j] = cnt[j] + plsc.all_reduce_population_count(
                            bidx == j
                        )
                for j in range(_K_CL):
                    for cc in range(nch):
                        snew[j, pl.ds(cc * L, L)] = jnp.zeros((L,), f32)

                def scat(m, carry):
                    asp = plsc.load_gather(prev, [jnp.full((L,), m, i32)])
                    for cc in range(nch):
                        cols = lane + (cc * L)
                        plsc.addupdate_scatter(
                            snew, [asp, cols], kbuf[m, pl.ds(cc * L, L)]
                        )
                    return carry

                lax.fori_loop(0, n, scat, jnp.int32(0))
                for j in range(_K_CL):
                    cntf = cnt[j].astype(f32)
                    nonz = cntf > 0.0
                    inv = 1.0 / jnp.maximum(cntf, 1.0)
                    dot = jnp.zeros((L,), f32)
                    for cc in range(nch):
                        sl = pl.ds(cc * L, L)
                        sn = snew[j, sl]
                        sacc[j, sl] = jnp.where(nonz, sn, sacc[j, sl])
                        oh = (prev[sl] == j).astype(f32)
                        wgt[j, sl] = jnp.where(nonz, oh * inv, wgt[j, sl])
                        dot = dot + oh * sn
                    tsum = dot
                    for sh in (8, 4, 2, 1):
                        t16[...] = tsum
                        tsum = tsum + plsc.load_gather(
                            t16, [jnp.bitwise_xor(lane, sh)]
                        )
                    c2n = tsum * inv * inv
                    c2b[j] = jnp.where(nonz, c2n, c2b[j][...])
                    cinv[j] = jnp.where(nonz, inv, cinv[j][...])
                return it + 1, jnp.any(changed)

            lax.while_loop(cond, step, (jnp.int32(0), jnp.bool_(True)))

            for cc in range(nch):
                sl = pl.ds(cc * L, L)
                acc = jnp.zeros((L,), f32)
                for j in range(_K_CL):
                    acc = acc + wgt[j, sl]
                ubuf[sl] = acc * (1.0 / _K_CL)
            pltpu.sync_copy(ubuf, u_hbm.at[cls])
            return _carry

        lax.fori_loop(0, per_w, run_class, jnp.int32(0))

    return body(K, idxp)


def _proto_body(u_ref, x_ref, w_ref, b_ref, p_ref):
    t = jnp.dot(u_ref[0], x_ref[0], preferred_element_type=jnp.float32)
    p_ref[0] = (
        jnp.dot(t, w_ref[...], preferred_element_type=jnp.float32)
        + b_ref[...]
    )


def _proto(u, x, W, b):
    n_way, n, d = x.shape
    out = pl.pallas_call(
        _proto_body,
        grid=(n_way,),
        in_specs=[
            pl.BlockSpec((1, 1, n), lambda c: (c, 0, 0)),
            pl.BlockSpec((1, n, d), lambda c: (c, 0, 0)),
            pl.BlockSpec((d, d), lambda c: (0, 0)),
            pl.BlockSpec((1, d), lambda c: (0, 0)),
        ],
        out_specs=pl.BlockSpec((1, 1, d), lambda c: (c, 0, 0)),
        out_shape=jax.ShapeDtypeStruct((n_way, 1, d), jnp.float32),
    )(u.reshape(n_way, 1, n), x, W, b.reshape(1, d))
    return out.reshape(n_way, d)


def _cdist_body(q_ref, w_ref, b_ref, p_ref, o_ref):
    q = q_ref[0]
    qe = (
        jnp.dot(q, w_ref[...], preferred_element_type=jnp.float32)
        + b_ref[...]
    )
    p = p_ref[...]
    q2 = jnp.sum(qe * qe, axis=1, keepdims=True)
    p2 = jnp.sum(p * p, axis=1, keepdims=True)
    qp = lax.dot_general(
        qe, p, (((1,), (1,)), ((), ())), preferred_element_type=jnp.float32
    )
    d2 = q2 + p2.T - 2.0 * qp
    o_ref[0] = -jnp.sqrt(jnp.maximum(d2, 1e-12))


def _cdist_logits(q, W, b, p):
    n_way, nq, d = q.shape
    return pl.pallas_call(
        _cdist_body,
        grid=(n_way,),
        in_specs=[
            pl.BlockSpec((1, nq, d), lambda c: (c, 0, 0)),
            pl.BlockSpec((d, d), lambda c: (0, 0)),
            pl.BlockSpec((1, d), lambda c: (0, 0)),
            pl.BlockSpec((n_way, d), lambda c: (0, 0)),
        ],
        out_specs=pl.BlockSpec((1, nq, n_way), lambda c: (c, 0, 0)),
        out_shape=jax.ShapeDtypeStruct((n_way, nq, n_way), jnp.float32),
    )(q, W, b.reshape(1, d), p)


@jax.jit
def _pipeline(support, query, W, b):
    n_way, n_support, d = support.shape

    kkey = jax.random.key(42)
    keys = jax.random.split(kkey, n_way)
    idx = jax.vmap(lambda k: jax.random.permutation(k, n_support)[:_K_CL])(keys)
    idxp = jnp.pad(idx.astype(jnp.int32), ((0, 0), (0, 16 - _K_CL)))

    K = _gram(support, W, b)
    u = _sc_kmeans(K, idxp)
    p = _proto(u, support, W, b)
    return _cdist_logits(query, W, b, p)


def kernel(support, query, W, b):
    return _pipeline(support, query, W, b)


# speedup vs baseline: 1.3431x; 1.3431x over previous
import jax
import jax.numpy as jnp
from jax import lax
from jax.experimental import pallas as pl
from jax.experimental.pallas import tpu as pltpu

_K_CL = 5
_K_PAD = 8
_MAX_ITER = 100


def _gram_body(x_ref, w_ref, b_ref, k_ref):
    cb, n, d = x_ref.shape
    x2d = x_ref[...].reshape(cb * n, d)
    emb = (
        jnp.dot(x2d, w_ref[...], preferred_element_type=jnp.float32)
        + b_ref[...]
    )
    for c in range(cb):
        ec = emb[c * n : (c + 1) * n]
        k_ref[c] = lax.dot_general(
            ec, ec, (((1,), (1,)), ((), ())), preferred_element_type=jnp.float32
        )


def _gram(x, W, b, class_block=8):
    n_way, n, d = x.shape
    return pl.pallas_call(
        _gram_body,
        grid=(n_way // class_block,),
        in_specs=[
            pl.BlockSpec((class_block, n, d), lambda i: (i, 0, 0)),
            pl.BlockSpec((d, d), lambda i: (0, 0)),
            pl.BlockSpec((1, d), lambda i: (0, 0)),
        ],
        out_specs=pl.BlockSpec((class_block, n, n), lambda i: (i, 0, 0)),
        out_shape=jax.ShapeDtypeStruct((n_way, n, n), jnp.float32),
    )(x, W, b.reshape(1, d))


def _kmeans_body(k_ref, w0_ref, u_ref, w_ref, prev_ref):
    n_way, kp, n = w0_ref.shape
    w_ref[...] = w0_ref[...]
    prev_ref[...] = jnp.full((n_way, n), -1, jnp.int32)

    jidx = lax.broadcasted_iota(jnp.int32, (kp, n), 0)
    jbad = jidx >= _K_CL

    def one_class(c, carry):
        kc = k_ref[c]

        def cond(it_ch):
            it, changed = it_ch
            return jnp.logical_and(it < _MAX_ITER, changed)

        def body(it_ch):
            it, _ = it_ch
            wc = w_ref[c]
            g = lax.dot_general(
                wc, kc, (((1,), (0,)), ((), ())),
                preferred_element_type=jnp.float32,
            )
            c2 = jnp.sum(g * wc, axis=1, keepdims=True)
            score = c2 - 2.0 * g
            score = jnp.where(jbad, jnp.inf, score)
            smin = jnp.min(score, axis=0, keepdims=True)
            assign = jnp.min(
                jnp.where(score == smin, jidx, _K_PAD), axis=0, keepdims=True
            )
            onehot = (jidx == assign).astype(jnp.float32)
            counts = jnp.sum(onehot, axis=1, keepdims=True)
            w_ref[c] = jnp.where(counts > 0.0, onehot / counts, wc)
            ch = jnp.any(assign[0] != prev_ref[c])
            prev_ref[c] = assign[0]
            return it + 1, ch

        lax.while_loop(cond, body, (jnp.int32(0), jnp.bool_(True)))
        return carry

    lax.fori_loop(0, n_way, one_class, jnp.int32(0))
    u_ref[...] = jnp.sum(w_ref[...], axis=1) * (1.0 / _K_CL)


def _kmeans(K, w0):
    n_way, kp, n = w0.shape
    return pl.pallas_call(
        _kmeans_body,
        in_specs=[
            pl.BlockSpec((n_way, n, n), lambda: (0, 0, 0)),
            pl.BlockSpec((n_way, kp, n), lambda: (0, 0, 0)),
        ],
        out_specs=pl.BlockSpec((n_way, n), lambda: (0, 0)),
        out_shape=jax.ShapeDtypeStruct((n_way, n), jnp.float32),
        scratch_shapes=[
            pltpu.VMEM((n_way, kp, n), jnp.float32),
            pltpu.VMEM((n_way, n), jnp.int32),
        ],
    )(K, w0)


def _proto_body(u_ref, x_ref, w_ref, b_ref, p_ref):
    t = jnp.dot(u_ref[0], x_ref[0], preferred_element_type=jnp.float32)
    p_ref[0] = (
        jnp.dot(t, w_ref[...], preferred_element_type=jnp.float32)
        + b_ref[...]
    )


def _proto(u, x, W, b):
    n_way, n, d = x.shape
    out = pl.pallas_call(
        _proto_body,
        grid=(n_way,),
        in_specs=[
            pl.BlockSpec((1, 1, n), lambda c: (c, 0, 0)),
            pl.BlockSpec((1, n, d), lambda c: (c, 0, 0)),
            pl.BlockSpec((d, d), lambda c: (0, 0)),
            pl.BlockSpec((1, d), lambda c: (0, 0)),
        ],
        out_specs=pl.BlockSpec((1, 1, d), lambda c: (c, 0, 0)),
        out_shape=jax.ShapeDtypeStruct((n_way, 1, d), jnp.float32),
    )(u.reshape(n_way, 1, n), x, W, b.reshape(1, d))
    return out.reshape(n_way, d)


def _cdist_body(q_ref, w_ref, b_ref, p_ref, o_ref):
    q = q_ref[0]
    qe = (
        jnp.dot(q, w_ref[...], preferred_element_type=jnp.float32)
        + b_ref[...]
    )
    p = p_ref[...]
    q2 = jnp.sum(qe * qe, axis=1, keepdims=True)
    p2 = jnp.sum(p * p, axis=1, keepdims=True)
    qp = lax.dot_general(
        qe, p, (((1,), (1,)), ((), ())), preferred_element_type=jnp.float32
    )
    d2 = q2 + p2.T - 2.0 * qp
    o_ref[0] = -jnp.sqrt(jnp.maximum(d2, 1e-12))


def _cdist_logits(q, W, b, p):
    n_way, nq, d = q.shape
    return pl.pallas_call(
        _cdist_body,
        grid=(n_way,),
        in_specs=[
            pl.BlockSpec((1, nq, d), lambda c: (c, 0, 0)),
            pl.BlockSpec((d, d), lambda c: (0, 0)),
            pl.BlockSpec((1, d), lambda c: (0, 0)),
            pl.BlockSpec((n_way, d), lambda c: (0, 0)),
        ],
        out_specs=pl.BlockSpec((1, nq, n_way), lambda c: (c, 0, 0)),
        out_shape=jax.ShapeDtypeStruct((n_way, nq, n_way), jnp.float32),
    )(q, W, b.reshape(1, d), p)


@jax.jit
def _pipeline(support, query, W, b):
    n_way, n_support, d = support.shape

    kkey = jax.random.key(42)
    keys = jax.random.split(kkey, n_way)
    idx = jax.vmap(lambda k: jax.random.permutation(k, n_support)[:_K_CL])(keys)
    jj = jnp.arange(_K_PAD)[None, :, None]
    nn = jnp.arange(n_support)[None, None, :]
    idx_pad = jnp.pad(idx, ((0, 0), (0, _K_PAD - _K_CL)), constant_values=-1)
    w0 = (nn == idx_pad[:, :, None]).astype(jnp.float32) * (jj < _K_CL)

    K = _gram(support, W, b)
    u = _kmeans(K, w0)
    p = _proto(u, support, W, b)
    return _cdist_logits(query, W, b, p)


def kernel(support, query, W, b):
    return _pipeline(support, query, W, b)
